# Initial kernel scaffold; baseline (speedup 1.0000x reference)
import jax
import jax.numpy as jnp
from jax.experimental import pallas as pl


def kernel(boxes, scores, labels):
    raise NotImplementedError("write your pallas kernel here")


# speedup vs baseline: 2.0500x; 2.0500x over previous
import jax
import jax.numpy as jnp
from jax.experimental import pallas as pl

IOU_THRESH = 0.5
SCORE_THRESH = 0.05
MAX_KEEP = 100

_R, _L = 160, 128
_NEG_INF = float("-inf")
_BIG_I32 = 2**31 - 1


def _nms_body(x1_ref, y1_ref, x2_ref, y2_ref, s_ref, l_ref, obox_ref, olab_ref):
    x1 = x1_ref[0]
    y1 = y1_ref[0]
    x2 = x2_ref[0]
    y2 = y2_ref[0]
    s = s_ref[0]
    lab = l_ref[0]
    area = (x2 - x1) * (y2 - y1)

    row = jax.lax.broadcasted_iota(jnp.int32, (_R, _L), 0)
    lane = jax.lax.broadcasted_iota(jnp.int32, (_R, _L), 1)
    fidx = row * _L + lane

    row1 = jax.lax.broadcasted_iota(jnp.int32, (8, _L), 0)
    lane1 = jax.lax.broadcasted_iota(jnp.int32, (8, _L), 1)

    obox_ref[0] = jnp.zeros((8, _L), jnp.float32)
    olab_ref[0] = jnp.zeros((8, _L), olab_ref.dtype)

    ws0 = jnp.where(s > SCORE_THRESH, s, _NEG_INF)

    def step(t, ws):
        m = jnp.max(ws)
        has = m > _NEG_INF
        idx = jnp.min(jnp.where(ws == m, fidx, _BIG_I32))
        sel = fidx == idx
        cx1 = jnp.sum(jnp.where(sel, x1, 0.0))
        cy1 = jnp.sum(jnp.where(sel, y1, 0.0))
        cx2 = jnp.sum(jnp.where(sel, x2, 0.0))
        cy2 = jnp.sum(jnp.where(sel, y2, 0.0))
        car = jnp.sum(jnp.where(sel, area, 0.0))
        cs = jnp.sum(jnp.where(sel, s, 0.0))
        cl = jnp.sum(jnp.where(sel, lab, 0))

        xx1 = jnp.maximum(cx1, x1)
        yy1 = jnp.maximum(cy1, y1)
        xx2 = jnp.minimum(cx2, x2)
        yy2 = jnp.minimum(cy2, y2)
        w = jnp.maximum(xx2 - xx1, 0.0)
        h = jnp.maximum(yy2 - yy1, 0.0)
        inter = w * h
        iou = inter / (car + area - inter)
        suppress = (iou > IOU_THRESH) | sel
        ws = jnp.where(has & suppress, _NEG_INF, ws)

        hit = (lane1 == t) & has
        vals = jnp.where(
            row1 == 0, cx1,
            jnp.where(row1 == 1, cy1,
                      jnp.where(row1 == 2, cx2,
                                jnp.where(row1 == 3, cy2, cs))))
        obox_ref[0] = jnp.where(hit, vals, obox_ref[0])
        olab_ref[0] = jnp.where(hit, cl, olab_ref[0])
        return ws

    jax.lax.fori_loop(0, MAX_KEEP, step, ws0)


def kernel(boxes, scores, labels):
    b, n = scores.shape
    pad = _R * _L - n
    x1 = jnp.pad(boxes[..., 0], ((0, 0), (0, pad))).reshape(b, _R, _L)
    y1 = jnp.pad(boxes[..., 1], ((0, 0), (0, pad))).reshape(b, _R, _L)
    x2 = jnp.pad(boxes[..., 2], ((0, 0), (0, pad))).reshape(b, _R, _L)
    y2 = jnp.pad(boxes[..., 3], ((0, 0), (0, pad))).reshape(b, _R, _L)
    sp = jnp.pad(scores, ((0, 0), (0, pad)), constant_values=-1.0).reshape(b, _R, _L)
    lp = jnp.pad(labels, ((0, 0), (0, pad))).reshape(b, _R, _L)

    in_spec = pl.BlockSpec((1, _R, _L), lambda i: (i, 0, 0))
    out_spec = pl.BlockSpec((1, 8, _L), lambda i: (i, 0, 0))
    obox, olab = pl.pallas_call(
        _nms_body,
        grid=(b,),
        in_specs=[in_spec] * 6,
        out_specs=[out_spec, out_spec],
        out_shape=[
            jax.ShapeDtypeStruct((b, 8, _L), jnp.float32),
            jax.ShapeDtypeStruct((b, 8, _L), labels.dtype),
        ],
    )(x1, y1, x2, y2, sp, lp)

    pb = jnp.moveaxis(obox[:, 0:4, :MAX_KEEP], 1, 2)
    ps = obox[:, 4, :MAX_KEEP]
    pl_out = olab[:, 0, :MAX_KEEP]
    return pb, ps, pl_out


# speedup vs baseline: 7.6031x; 3.7089x over previous
import functools

import jax
import jax.numpy as jnp
from jax import lax
from jax.experimental import pallas as pl
from jax.experimental.pallas import tpu as pltpu
from jax.experimental.pallas import tpu_sc as plsc

IOU_THRESH = 0.5
SCORE_THRESH = 0.05
MAX_KEEP = 100

_NP = 20480
_G = 64
_NG = _NP // _G
_NG2 = _NG // 16
_NEG = float("-inf")


def _splat_f(x):
    return jnp.full((16,), x, jnp.float32)


def _splat_i(x):
    return jnp.full((16,), x, jnp.int32)


def _sc_nms(x1h, y1h, x2h, y2h, sh, lh, outf, outl,
            sc_v, x1_v, y1_v, x2_v, y2_v, lab_v, l1_v, l2_v,
            kx1_v, ky1_v, kx2_v, ky2_v, kar_v, osc_v, olb_v):
    wid = lax.axis_index("s") * 2 + lax.axis_index("c")

    @pl.when(wid < 4)
    def _():
        iota = lax.iota(jnp.int32, 16)
        mask0 = iota == 0

        pltpu.sync_copy(sh.at[wid], sc_v)
        pltpu.sync_copy(x1h.at[wid], x1_v)
        pltpu.sync_copy(y1h.at[wid], y1_v)
        pltpu.sync_copy(x2h.at[wid], x2_v)
        pltpu.sync_copy(y2h.at[wid], y2_v)
        pltpu.sync_copy(lh.at[wid], lab_v)

        zf = jnp.zeros((16,), jnp.float32)
        zi = jnp.zeros((16,), jnp.int32)
        for r in range(8):
            kx1_v[pl.ds(r * 16, 16)] = zf
            ky1_v[pl.ds(r * 16, 16)] = zf
            kx2_v[pl.ds(r * 16, 16)] = zf
            ky2_v[pl.ds(r * 16, 16)] = zf
            kar_v[pl.ds(r * 16, 16)] = zf
            osc_v[pl.ds(r * 16, 16)] = zf
            olb_v[pl.ds(r * 16, 16)] = zi
        l2_v[pl.ds(0, 16)] = _splat_f(_NEG)
        l2_v[pl.ds(16, 16)] = _splat_f(_NEG)

        def build1(g, carry):
            base = g * _G
            acc = _splat_f(_NEG)
            for k in range(_G // 16):
                idx = base + k * 16 + iota
                v = plsc.load_gather(sc_v, [idx])
                va = jnp.where(v > SCORE_THRESH, v, _NEG)
                plsc.store_scatter(sc_v, [idx], va)
                acc = jnp.maximum(acc, va)
            plsc.store_scatter(l1_v, [_splat_i(0) + g], _splat_f(jnp.max(acc)), mask=mask0)
            return carry

        lax.fori_loop(0, _NG, build1, 0)

        def build2(j, carry):
            v = plsc.load_gather(l1_v, [j * 16 + iota])
            plsc.store_scatter(l2_v, [_splat_i(0) + j], _splat_f(jnp.max(v)), mask=mask0)
            return carry

        lax.fori_loop(0, _NG2, build2, 0)

        def cond(carry):
            count, done = carry
            return jnp.logical_and(count < MAX_KEEP, jnp.logical_not(done))

        def body(carry):
            count, _ = carry
            la = l2_v[pl.ds(0, 16)]
            lb = l2_v[pl.ds(16, 16)]
            ma = jnp.max(la)
            m = jnp.maximum(ma, jnp.max(lb))
            has = m > _NEG
            pa = jnp.min(jnp.where(la == m, iota, 16))
            pb = jnp.min(jnp.where(lb == m, iota, 16))
            g2 = jnp.where(ma >= m, pa, 16 + pb)
            l1v = plsc.load_gather(l1_v, [g2 * 16 + iota])
            g1 = g2 * 16 + jnp.min(jnp.where(l1v == m, iota, 16))
            base = g1 * _G
            pos = base
            for k in range(_G // 16 - 1, -1, -1):
                sv = plsc.load_gather(sc_v, [base + k * 16 + iota])
                pk = jnp.min(jnp.where(sv == m, iota, 16))
                pos = jnp.where(pk < 16, base + k * 16 + pk, pos)
            idxv = _splat_i(0) + pos
            cx1 = plsc.load_gather(x1_v, [idxv])
            cy1 = plsc.load_gather(y1_v, [idxv])
            cx2 = plsc.load_gather(x2_v, [idxv])
            cy2 = plsc.load_gather(y2_v, [idxv])
            clb = plsc.load_gather(lab_v, [idxv])
            car = (cx2 - cx1) * (cy2 - cy1)

            sup = jnp.zeros((16,), jnp.bool_)
            for r in range(8):
                kx1 = kx1_v[pl.ds(r * 16, 16)]
                ky1 = ky1_v[pl.ds(r * 16, 16)]
                kx2 = kx2_v[pl.ds(r * 16, 16)]
                ky2 = ky2_v[pl.ds(r * 16, 16)]
                kar = kar_v[pl.ds(r * 16, 16)]
                xx1 = jnp.maximum(kx1, cx1)
                yy1 = jnp.maximum(ky1, cy1)
                xx2 = jnp.minimum(kx2, cx2)
                yy2 = jnp.minimum(ky2, cy2)
                w = jnp.maximum(xx2 - xx1, 0.0)
                h = jnp.maximum(yy2 - yy1, 0.0)
                inter = w * h
                iou = inter / (kar + car - inter)
                sup = jnp.logical_or(sup, iou > IOU_THRESH)
            ok = jnp.logical_and(has, jnp.logical_not(jnp.any(sup)))

            @pl.when(ok)
            def _():
                cidx = _splat_i(0) + count
                plsc.store_scatter(kx1_v, [cidx], cx1, mask=mask0)
                plsc.store_scatter(ky1_v, [cidx], cy1, mask=mask0)
                plsc.store_scatter(kx2_v, [cidx], cx2, mask=mask0)
                plsc.store_scatter(ky2_v, [cidx], cy2, mask=mask0)
                plsc.store_scatter(kar_v, [cidx], car, mask=mask0)
                plsc.store_scatter(osc_v, [cidx], _splat_f(m), mask=mask0)
                plsc.store_scatter(olb_v, [cidx], clb, mask=mask0)

            plsc.store_scatter(sc_v, [idxv], _splat_f(_NEG), mask=mask0)
            acc = _splat_f(_NEG)
            for k in range(_G // 16):
                acc = jnp.maximum(acc, plsc.load_gather(sc_v, [base + k * 16 + iota]))
            plsc.store_scatter(l1_v, [_splat_i(0) + g1], _splat_f(jnp.max(acc)), mask=mask0)
            l1v2 = plsc.load_gather(l1_v, [g2 * 16 + iota])
            plsc.store_scatter(l2_v, [_splat_i(0) + g2], _splat_f(jnp.max(l1v2)), mask=mask0)

            return count + jnp.where(ok, 1, 0), jnp.logical_not(has)

        lax.while_loop(cond, body, (jnp.int32(0), jnp.bool_(False)))

        for r, ref in enumerate((kx1_v, ky1_v, kx2_v, ky2_v, osc_v)):
            pltpu.sync_copy(ref, outf.at[pl.ds((wid * 5 + r) * 128, 128)])
        pltpu.sync_copy(olb_v, outl.at[pl.ds(wid * 128, 128)])


def _sc_call(x1, y1, x2, y2, s, lab):
    mesh = plsc.VectorSubcoreMesh(core_axis_name="c", subcore_axis_name="s")
    f = pl.kernel(
        _sc_nms,
        out_type=[
            jax.ShapeDtypeStruct((4 * 5 * 128,), jnp.float32),
            jax.ShapeDtypeStruct((4 * 128,), jnp.int32),
        ],
        mesh=mesh,
        compiler_params=pltpu.CompilerParams(needs_layout_passes=False),
        scratch_types=[
            pltpu.VMEM((_NP,), jnp.float32),
            pltpu.VMEM((_NP,), jnp.float32),
            pltpu.VMEM((_NP,), jnp.float32),
            pltpu.VMEM((_NP,), jnp.float32),
            pltpu.VMEM((_NP,), jnp.float32),
            pltpu.VMEM((_NP,), jnp.int32),
            pltpu.VMEM((_NG,), jnp.float32),
            pltpu.VMEM((32,), jnp.float32),
            pltpu.VMEM((128,), jnp.float32),
            pltpu.VMEM((128,), jnp.float32),
            pltpu.VMEM((128,), jnp.float32),
            pltpu.VMEM((128,), jnp.float32),
            pltpu.VMEM((128,), jnp.float32),
            pltpu.VMEM((128,), jnp.float32),
            pltpu.VMEM((128,), jnp.int32),
        ],
    )
    return f(x1, y1, x2, y2, s, lab)


def kernel(boxes, scores, labels):
    b, n = scores.shape
    pad = _NP - n
    x1 = jnp.pad(boxes[..., 0], ((0, 0), (0, pad)))
    y1 = jnp.pad(boxes[..., 1], ((0, 0), (0, pad)))
    x2 = jnp.pad(boxes[..., 2], ((0, 0), (0, pad)))
    y2 = jnp.pad(boxes[..., 3], ((0, 0), (0, pad)))
    sp = jnp.pad(scores, ((0, 0), (0, pad)), constant_values=-1.0)
    lp = jnp.pad(labels, ((0, 0), (0, pad))).astype(jnp.int32)
    outf, outl = _sc_call(x1, y1, x2, y2, sp, lp)
    outf = outf.reshape(b, 5, 128)
    pb = jnp.moveaxis(outf[:, 0:4, :MAX_KEEP], 1, 2)
    ps = outf[:, 4, :MAX_KEEP]
    plb = outl.reshape(b, 128)[:, :MAX_KEEP].astype(labels.dtype)
    return pb, ps, plb


# speedup vs baseline: 10.5533x; 1.3880x over previous
import jax
import jax.numpy as jnp
from jax import lax
from jax.experimental import pallas as pl
from jax.experimental.pallas import tpu as pltpu
from jax.experimental.pallas import tpu_sc as plsc

IOU_THRESH = 0.5
SCORE_THRESH = 0.05
MAX_KEEP = 100

_NP = 20480
_G = 64
_NG = _NP // _G
_NG2 = _NG // 16
_NEG = float("-inf")


def _splat_f(x):
    return jnp.full((16,), x, jnp.float32)


def _splat_i(x):
    return jnp.full((16,), x, jnp.int32)


def _sc_nms(x1h, y1h, x2h, y2h, sh, lh, outf, outl,
            sc_v, x1_v, y1_v, x2_v, y2_v, lab_v, l1_v, l2_v,
            kx1_v, ky1_v, kx2_v, ky2_v, kar_v, osc_v, olb_v,
            s0, s1, s2, s3, s4, s5):
    wid = lax.axis_index("s") * 2 + lax.axis_index("c")

    @pl.when(wid < 4)
    def _():
        iota = lax.iota(jnp.int32, 16)
        mask0 = iota == 0

        c0 = pltpu.async_copy(sh.at[wid], sc_v, s0)
        c1 = pltpu.async_copy(x1h.at[wid], x1_v, s1)
        c2 = pltpu.async_copy(y1h.at[wid], y1_v, s2)
        c3 = pltpu.async_copy(x2h.at[wid], x2_v, s3)
        c4 = pltpu.async_copy(y2h.at[wid], y2_v, s4)
        c5 = pltpu.async_copy(lh.at[wid], lab_v, s5)

        zf = jnp.zeros((16,), jnp.float32)
        zi = jnp.zeros((16,), jnp.int32)
        for r in range(8):
            kx1_v[pl.ds(r * 16, 16)] = zf
            ky1_v[pl.ds(r * 16, 16)] = zf
            kx2_v[pl.ds(r * 16, 16)] = zf
            ky2_v[pl.ds(r * 16, 16)] = zf
            kar_v[pl.ds(r * 16, 16)] = zf
            osc_v[pl.ds(r * 16, 16)] = zf
            olb_v[pl.ds(r * 16, 16)] = zi
        l2_v[pl.ds(0, 16)] = _splat_f(_NEG)
        l2_v[pl.ds(16, 16)] = _splat_f(_NEG)

        c0.wait()

        @plsc.parallel_loop(0, _NG, 1, unroll=4)
        def _build1(g):
            base = g * _G
            v0 = plsc.load_gather(sc_v, [base + iota])
            v1 = plsc.load_gather(sc_v, [base + 16 + iota])
            v2 = plsc.load_gather(sc_v, [base + 32 + iota])
            v3 = plsc.load_gather(sc_v, [base + 48 + iota])
            acc = jnp.maximum(jnp.maximum(v0, v1), jnp.maximum(v2, v3))
            plsc.store_scatter(l1_v, [_splat_i(0) + g], _splat_f(jnp.max(acc)),
                               mask=mask0)

        @plsc.parallel_loop(0, _NG2, 1, unroll=2)
        def _build2(j):
            v = plsc.load_gather(l1_v, [j * 16 + iota])
            plsc.store_scatter(l2_v, [_splat_i(0) + j], _splat_f(jnp.max(v)),
                               mask=mask0)

        c1.wait()
        c2.wait()
        c3.wait()
        c4.wait()
        c5.wait()

        def cond(carry):
            count, done = carry
            return jnp.logical_and(count < MAX_KEEP, jnp.logical_not(done))

        def body(carry):
            count, _ = carry
            la = l2_v[pl.ds(0, 16)]
            lb = l2_v[pl.ds(16, 16)]
            m = jnp.max(jnp.maximum(la, lb))
            has = m > SCORE_THRESH
            mv = _splat_f(0.0) + m
            fa = plsc.all_reduce_ffs(la == mv)
            fb = plsc.all_reduce_ffs(lb == mv)
            g2v = jnp.where(fa < 16, fa, 16 + fb)
            l1v = plsc.load_gather(l1_v, [g2v * 16 + iota])
            j1 = plsc.all_reduce_ffs(l1v == mv)
            g1v = g2v * 16 + j1
            basev = g1v * _G
            idx0 = basev + iota
            idx1 = basev + 16 + iota
            idx2 = basev + 32 + iota
            idx3 = basev + 48 + iota
            sv0 = plsc.load_gather(sc_v, [idx0])
            sv1 = plsc.load_gather(sc_v, [idx1])
            sv2 = plsc.load_gather(sc_v, [idx2])
            sv3 = plsc.load_gather(sc_v, [idx3])
            f0 = plsc.all_reduce_ffs(sv0 == mv)
            f1 = plsc.all_reduce_ffs(sv1 == mv)
            f2 = plsc.all_reduce_ffs(sv2 == mv)
            f3 = plsc.all_reduce_ffs(sv3 == mv)
            off = jnp.where(f0 < 16, f0,
                            jnp.where(f1 < 16, 16 + f1,
                                      jnp.where(f2 < 16, 32 + f2, 48 + f3)))
            idxv = basev + off
            cx1 = plsc.load_gather(x1_v, [idxv])
            cy1 = plsc.load_gather(y1_v, [idxv])
            cx2 = plsc.load_gather(x2_v, [idxv])
            cy2 = plsc.load_gather(y2_v, [idxv])
            clb = plsc.load_gather(lab_v, [idxv])
            car = (cx2 - cx1) * (cy2 - cy1)

            sup = jnp.zeros((16,), jnp.bool_)
            for r in range(8):
                kx1 = kx1_v[pl.ds(r * 16, 16)]
                ky1 = ky1_v[pl.ds(r * 16, 16)]
                kx2 = kx2_v[pl.ds(r * 16, 16)]
                ky2 = ky2_v[pl.ds(r * 16, 16)]
                kar = kar_v[pl.ds(r * 16, 16)]
                xx1 = jnp.maximum(kx1, cx1)
                yy1 = jnp.maximum(ky1, cy1)
                xx2 = jnp.minimum(kx2, cx2)
                yy2 = jnp.minimum(ky2, cy2)
                w = jnp.maximum(xx2 - xx1, 0.0)
                h = jnp.maximum(yy2 - yy1, 0.0)
                inter = w * h
                iou = inter / (kar + car - inter)
                sup = jnp.logical_or(sup, iou > IOU_THRESH)
            ok = jnp.logical_and(has, jnp.logical_not(jnp.any(sup)))

            @pl.when(ok)
            def _():
                cidx = _splat_i(0) + count
                plsc.store_scatter(kx1_v, [cidx], cx1, mask=mask0)
                plsc.store_scatter(ky1_v, [cidx], cy1, mask=mask0)
                plsc.store_scatter(kx2_v, [cidx], cx2, mask=mask0)
                plsc.store_scatter(ky2_v, [cidx], cy2, mask=mask0)
                plsc.store_scatter(kar_v, [cidx], car, mask=mask0)
                plsc.store_scatter(osc_v, [cidx], _splat_f(0.0) + m, mask=mask0)
                plsc.store_scatter(olb_v, [cidx], clb, mask=mask0)

            plsc.store_scatter(sc_v, [idxv], _splat_f(_NEG), mask=mask0)
            n0 = jnp.where(idx0 == idxv, _NEG, sv0)
            n1 = jnp.where(idx1 == idxv, _NEG, sv1)
            n2 = jnp.where(idx2 == idxv, _NEG, sv2)
            n3 = jnp.where(idx3 == idxv, _NEG, sv3)
            gm = jnp.max(jnp.maximum(jnp.maximum(n0, n1), jnp.maximum(n2, n3)))
            plsc.store_scatter(l1_v, [g1v], _splat_f(0.0) + gm, mask=mask0)
            nl1 = jnp.where(iota == j1, gm, l1v)
            plsc.store_scatter(l2_v, [g2v], _splat_f(jnp.max(nl1)), mask=mask0)

            return count + jnp.where(ok, 1, 0), jnp.logical_not(has)

        lax.while_loop(cond, body, (jnp.int32(0), jnp.bool_(False)))

        for r, ref in enumerate((kx1_v, ky1_v, kx2_v, ky2_v, osc_v)):
            pltpu.sync_copy(ref, outf.at[pl.ds((wid * 5 + r) * 128, 128)])
        pltpu.sync_copy(olb_v, outl.at[pl.ds(wid * 128, 128)])


def _sc_call(x1, y1, x2, y2, s, lab):
    mesh = plsc.VectorSubcoreMesh(core_axis_name="c", subcore_axis_name="s")
    f = pl.kernel(
        _sc_nms,
        out_type=[
            jax.ShapeDtypeStruct((4 * 5 * 128,), jnp.float32),
            jax.ShapeDtypeStruct((4 * 128,), jnp.int32),
        ],
        mesh=mesh,
        compiler_params=pltpu.CompilerParams(needs_layout_passes=False),
        scratch_types=[
            pltpu.VMEM((_NP,), jnp.float32),
            pltpu.VMEM((_NP,), jnp.float32),
            pltpu.VMEM((_NP,), jnp.float32),
            pltpu.VMEM((_NP,), jnp.float32),
            pltpu.VMEM((_NP,), jnp.float32),
            pltpu.VMEM((_NP,), jnp.int32),
            pltpu.VMEM((_NG,), jnp.float32),
            pltpu.VMEM((32,), jnp.float32),
            pltpu.VMEM((128,), jnp.float32),
            pltpu.VMEM((128,), jnp.float32),
            pltpu.VMEM((128,), jnp.float32),
            pltpu.VMEM((128,), jnp.float32),
            pltpu.VMEM((128,), jnp.float32),
            pltpu.VMEM((128,), jnp.float32),
            pltpu.VMEM((128,), jnp.int32),
            pltpu.SemaphoreType.DMA,
            pltpu.SemaphoreType.DMA,
            pltpu.SemaphoreType.DMA,
            pltpu.SemaphoreType.DMA,
            pltpu.SemaphoreType.DMA,
            pltpu.SemaphoreType.DMA,
        ],
    )
    return f(x1, y1, x2, y2, s, lab)


def kernel(boxes, scores, labels):
    b, n = scores.shape
    pad = _NP - n
    x1 = jnp.pad(boxes[..., 0], ((0, 0), (0, pad)))
    y1 = jnp.pad(boxes[..., 1], ((0, 0), (0, pad)))
    x2 = jnp.pad(boxes[..., 2], ((0, 0), (0, pad)))
    y2 = jnp.pad(boxes[..., 3], ((0, 0), (0, pad)))
    sp = jnp.pad(scores, ((0, 0), (0, pad)), constant_values=-1.0)
    lp = jnp.pad(labels, ((0, 0), (0, pad))).astype(jnp.int32)
    outf, outl = _sc_call(x1, y1, x2, y2, sp, lp)
    outf = outf.reshape(b, 5, 128)
    pb = jnp.moveaxis(outf[:, 0:4, :MAX_KEEP], 1, 2)
    ps = outf[:, 4, :MAX_KEEP]
    plb = outl.reshape(b, 128)[:, :MAX_KEEP].astype(labels.dtype)
    return pb, ps, plb


# speedup vs baseline: 11.0440x; 1.0465x over previous
import jax
import jax.numpy as jnp
from jax import lax
from jax.experimental import pallas as pl
from jax.experimental.pallas import tpu as pltpu
from jax.experimental.pallas import tpu_sc as plsc

IOU_THRESH = 0.5
SCORE_THRESH = 0.05
MAX_KEEP = 100

_NP = 20480
_G = 64
_NG = _NP // _G
_NG2 = _NG // 16
_NEG = float("-inf")


def _splat_f(x):
    return jnp.full((16,), x, jnp.float32)


def _splat_i(x):
    return jnp.full((16,), x, jnp.int32)


def _sc_nms(x1h, y1h, x2h, y2h, sh, lh, outf, outl,
            sc_v, x1_v, y1_v, x2_v, y2_v, lab_v, l1_v, l2_v,
            kx1_v, ky1_v, kx2_v, ky2_v, kar_v, osc_v, olb_v,
            s0, s1, s2, s3, s4, s5):
    wid = lax.axis_index("s")

    @pl.when(wid < 4)
    def _():
        iota = lax.iota(jnp.int32, 16)
        mask0 = iota == 0

        c0 = pltpu.async_copy(sh.at[wid], sc_v, s0)
        c1 = pltpu.async_copy(x1h.at[wid], x1_v, s1)
        c2 = pltpu.async_copy(y1h.at[wid], y1_v, s2)
        c3 = pltpu.async_copy(x2h.at[wid], x2_v, s3)
        c4 = pltpu.async_copy(y2h.at[wid], y2_v, s4)
        c5 = pltpu.async_copy(lh.at[wid], lab_v, s5)

        zf = jnp.zeros((16,), jnp.float32)
        zi = jnp.zeros((16,), jnp.int32)
        for r in range(8):
            kx1_v[pl.ds(r * 16, 16)] = zf
            ky1_v[pl.ds(r * 16, 16)] = zf
            kx2_v[pl.ds(r * 16, 16)] = zf
            ky2_v[pl.ds(r * 16, 16)] = zf
            kar_v[pl.ds(r * 16, 16)] = zf
            osc_v[pl.ds(r * 16, 16)] = zf
            olb_v[pl.ds(r * 16, 16)] = zi
        l2_v[pl.ds(0, 16)] = _splat_f(_NEG)
        l2_v[pl.ds(16, 16)] = _splat_f(_NEG)

        c0.wait()

        @plsc.parallel_loop(0, _NG, 1, unroll=4)
        def _build1(g):
            base = g * _G
            v0 = plsc.load_gather(sc_v, [base + iota])
            v1 = plsc.load_gather(sc_v, [base + 16 + iota])
            v2 = plsc.load_gather(sc_v, [base + 32 + iota])
            v3 = plsc.load_gather(sc_v, [base + 48 + iota])
            acc = jnp.maximum(jnp.maximum(v0, v1), jnp.maximum(v2, v3))
            plsc.store_scatter(l1_v, [_splat_i(0) + g], _splat_f(jnp.max(acc)),
                               mask=mask0)

        @plsc.parallel_loop(0, _NG2, 1, unroll=2)
        def _build2(j):
            v = plsc.load_gather(l1_v, [j * 16 + iota])
            plsc.store_scatter(l2_v, [_splat_i(0) + j], _splat_f(jnp.max(v)),
                               mask=mask0)

        c1.wait()
        c2.wait()
        c3.wait()
        c4.wait()
        c5.wait()

        def cond(carry):
            count, done = carry
            return jnp.logical_and(count < MAX_KEEP, jnp.logical_not(done))

        def body(carry):
            count, _ = carry
            la = l2_v[pl.ds(0, 16)]
            lb = l2_v[pl.ds(16, 16)]
            m = jnp.max(jnp.maximum(la, lb))
            has = m > SCORE_THRESH
            mv = _splat_f(0.0) + m
            fa = plsc.all_reduce_ffs(la == mv)
            fb = plsc.all_reduce_ffs(lb == mv)
            g2v = jnp.where(fa < 16, fa, 16 + fb)
            l1v = plsc.load_gather(l1_v, [g2v * 16 + iota])
            j1 = plsc.all_reduce_ffs(l1v == mv)
            g1v = g2v * 16 + j1
            basev = g1v * _G
            idx0 = basev + iota
            idx1 = basev + 16 + iota
            idx2 = basev + 32 + iota
            idx3 = basev + 48 + iota
            sv0 = plsc.load_gather(sc_v, [idx0])
            sv1 = plsc.load_gather(sc_v, [idx1])
            sv2 = plsc.load_gather(sc_v, [idx2])
            sv3 = plsc.load_gather(sc_v, [idx3])
            f0 = plsc.all_reduce_ffs(sv0 == mv)
            f1 = plsc.all_reduce_ffs(sv1 == mv)
            f2 = plsc.all_reduce_ffs(sv2 == mv)
            f3 = plsc.all_reduce_ffs(sv3 == mv)
            off = jnp.where(f0 < 16, f0,
                            jnp.where(f1 < 16, 16 + f1,
                                      jnp.where(f2 < 16, 32 + f2, 48 + f3)))
            idxv = basev + off
            cx1 = plsc.load_gather(x1_v, [idxv])
            cy1 = plsc.load_gather(y1_v, [idxv])
            cx2 = plsc.load_gather(x2_v, [idxv])
            cy2 = plsc.load_gather(y2_v, [idxv])
            clb = plsc.load_gather(lab_v, [idxv])
            car = (cx2 - cx1) * (cy2 - cy1)

            sup = jnp.zeros((16,), jnp.bool_)
            for r in range(8):
                kx1 = kx1_v[pl.ds(r * 16, 16)]
                ky1 = ky1_v[pl.ds(r * 16, 16)]
                kx2 = kx2_v[pl.ds(r * 16, 16)]
                ky2 = ky2_v[pl.ds(r * 16, 16)]
                kar = kar_v[pl.ds(r * 16, 16)]
                xx1 = jnp.maximum(kx1, cx1)
                yy1 = jnp.maximum(ky1, cy1)
                xx2 = jnp.minimum(kx2, cx2)
                yy2 = jnp.minimum(ky2, cy2)
                w = jnp.maximum(xx2 - xx1, 0.0)
                h = jnp.maximum(yy2 - yy1, 0.0)
                inter = w * h
                iou = inter / (kar + car - inter)
                sup = jnp.logical_or(sup, iou > IOU_THRESH)
            ok = jnp.logical_and(has, jnp.logical_not(jnp.any(sup)))

            @pl.when(ok)
            def _():
                cidx = _splat_i(0) + count
                plsc.store_scatter(kx1_v, [cidx], cx1, mask=mask0)
                plsc.store_scatter(ky1_v, [cidx], cy1, mask=mask0)
                plsc.store_scatter(kx2_v, [cidx], cx2, mask=mask0)
                plsc.store_scatter(ky2_v, [cidx], cy2, mask=mask0)
                plsc.store_scatter(kar_v, [cidx], car, mask=mask0)
                plsc.store_scatter(osc_v, [cidx], _splat_f(0.0) + m, mask=mask0)
                plsc.store_scatter(olb_v, [cidx], clb, mask=mask0)

            plsc.store_scatter(sc_v, [idxv], _splat_f(_NEG), mask=mask0)
            n0 = jnp.where(idx0 == idxv, _NEG, sv0)
            n1 = jnp.where(idx1 == idxv, _NEG, sv1)
            n2 = jnp.where(idx2 == idxv, _NEG, sv2)
            n3 = jnp.where(idx3 == idxv, _NEG, sv3)
            gm = jnp.max(jnp.maximum(jnp.maximum(n0, n1), jnp.maximum(n2, n3)))
            plsc.store_scatter(l1_v, [g1v], _splat_f(0.0) + gm, mask=mask0)
            nl1 = jnp.where(iota == j1, gm, l1v)
            plsc.store_scatter(l2_v, [g2v], _splat_f(jnp.max(nl1)), mask=mask0)

            return count + jnp.where(ok, 1, 0), jnp.logical_not(has)

        lax.while_loop(cond, body, (jnp.int32(0), jnp.bool_(False)))

        for r, ref in enumerate((kx1_v, ky1_v, kx2_v, ky2_v, osc_v)):
            pltpu.sync_copy(ref, outf.at[pl.ds((wid * 5 + r) * 128, 128)])
        pltpu.sync_copy(olb_v, outl.at[pl.ds(wid * 128, 128)])


def _sc_call(x1, y1, x2, y2, s, lab):
    mesh = plsc.VectorSubcoreMesh(core_axis_name="c", subcore_axis_name="s",
                                  num_cores=1)
    f = pl.kernel(
        _sc_nms,
        out_type=[
            jax.ShapeDtypeStruct((4 * 5 * 128,), jnp.float32),
            jax.ShapeDtypeStruct((4 * 128,), jnp.int32),
        ],
        mesh=mesh,
        compiler_params=pltpu.CompilerParams(needs_layout_passes=False),
        scratch_types=[
            pltpu.VMEM((_NP,), jnp.float32),
            pltpu.VMEM((_NP,), jnp.float32),
            pltpu.VMEM((_NP,), jnp.float32),
            pltpu.VMEM((_NP,), jnp.float32),
            pltpu.VMEM((_NP,), jnp.float32),
            pltpu.VMEM((_NP,), jnp.int32),
            pltpu.VMEM((_NG,), jnp.float32),
            pltpu.VMEM((32,), jnp.float32),
            pltpu.VMEM((128,), jnp.float32),
            pltpu.VMEM((128,), jnp.float32),
            pltpu.VMEM((128,), jnp.float32),
            pltpu.VMEM((128,), jnp.float32),
            pltpu.VMEM((128,), jnp.float32),
            pltpu.VMEM((128,), jnp.float32),
            pltpu.VMEM((128,), jnp.int32),
            pltpu.SemaphoreType.DMA,
            pltpu.SemaphoreType.DMA,
            pltpu.SemaphoreType.DMA,
            pltpu.SemaphoreType.DMA,
            pltpu.SemaphoreType.DMA,
            pltpu.SemaphoreType.DMA,
        ],
    )
    return f(x1, y1, x2, y2, s, lab)


def kernel(boxes, scores, labels):
    b, n = scores.shape
    pad = _NP - n
    x1 = jnp.pad(boxes[..., 0], ((0, 0), (0, pad)))
    y1 = jnp.pad(boxes[..., 1], ((0, 0), (0, pad)))
    x2 = jnp.pad(boxes[..., 2], ((0, 0), (0, pad)))
    y2 = jnp.pad(boxes[..., 3], ((0, 0), (0, pad)))
    sp = jnp.pad(scores, ((0, 0), (0, pad)), constant_values=-1.0)
    lp = jnp.pad(labels, ((0, 0), (0, pad))).astype(jnp.int32)
    outf, outl = _sc_call(x1, y1, x2, y2, sp, lp)
    outf = outf.reshape(b, 5, 128)
    pb = jnp.moveaxis(outf[:, 0:4, :MAX_KEEP], 1, 2)
    ps = outf[:, 4, :MAX_KEEP]
    plb = outl.reshape(b, 128)[:, :MAX_KEEP].astype(labels.dtype)
    return pb, ps, plb
